# baseline (speedup 1.0000x reference)
import functools

import jax
import jax.numpy as jnp
from jax import lax
from jax.experimental import pallas as pl
from jax.experimental.pallas import tpu as pltpu
from jax.experimental.pallas import tpu_sc as plsc

D = 768
E = 8
C = 128
T = 16384
H = 3072
TB = 1024
NTILES = 32
TPW = T // NTILES
CHUNK = 128
CCH = 64
SPT = (E * C) // NTILES
SC_SLOTS = (E * C) // 2
ZR = 64
DUMMY_BASE = 1032
FFN_ROWS = 1152

@functools.lru_cache(maxsize=None)
def _mesh():
    return plsc.VectorSubcoreMesh(core_axis_name="c", subcore_axis_name="s")


def _rs_body(h_ref, w_ref, probs_ref, pp_ref, ei_ref, g_ref, loss_ref, pt_ref):
    i = pl.program_id(0)
    NB = T // TB

    @pl.when(i < NB)
    def _():
        x = h_ref[...]
        w = w_ref[...]
        lt = lax.dot_general(w, x, (((0,), (1,)), ((), ())),
                             preferred_element_type=jnp.float32)
        m = jnp.max(lt, axis=0, keepdims=True)
        ex = jnp.exp(lt - m)
        pt = ex / jnp.sum(ex, axis=0, keepdims=True)
        probs_ref[...] = pt.T
        pt_ref[:, pl.ds(jnp.minimum(i, NB - 1), 1), :] = pt[:, None, :]

    @pl.when(i == NB)
    def _():
        RR, CC = T // 1024, 1024
        pe = [pt_ref[e] for e in range(E)]
        gate = pe[0]
        for e in range(1, E):
            gate = jnp.maximum(gate, pe[e])
        idx = jnp.full((RR, CC), E, jnp.int32)
        for e in range(E - 1, -1, -1):
            idx = jnp.where(pe[e] == gate, e, idx)

        r_iota = lax.broadcasted_iota(jnp.int32, (CC, CC), 0)
        c_iota = lax.broadcasted_iota(jnp.int32, (CC, CC), 1)
        u_inc = (r_iota <= c_iota).astype(jnp.float32)
        rs_iota = lax.broadcasted_iota(jnp.int32, (RR, RR), 0)
        cs_iota = lax.broadcasted_iota(jnp.int32, (RR, RR), 1)
        l_exc = (rs_iota > cs_iota).astype(jnp.float32)

        def cum_excl(x):
            incl = jnp.dot(x, u_inc, preferred_element_type=jnp.float32)
            rowsum = jnp.sum(x, axis=1, keepdims=True)
            off = jnp.dot(l_exc, rowsum, preferred_element_type=jnp.float32)
            return incl + off - x

        bits = lax.bitcast_convert_type(gate, jnp.int32)
        ri = lax.broadcasted_iota(jnp.int32, (RR, CC), 0)
        ci = lax.broadcasted_iota(jnp.int32, (RR, CC), 1)
        t_iota = ri * CC + ci
        pp = DUMMY_BASE + jnp.bitwise_and(t_iota, 63)
        loss = jnp.float32(0.0)
        for e in range(E):
            ohe = (idx == e).astype(jnp.float32)
            loss = loss + jnp.sum(ohe) * jnp.sum(pe[e])

            def bs(_, carry):
                lo, hi = carry
                mid = lax.div(lo + hi, jnp.int32(2))
                cnt = jnp.sum(ohe * (bits >= mid).astype(jnp.float32))
                take = cnt >= C
                return (jnp.where(take, mid, lo), jnp.where(take, hi, mid))

            thr, _ = lax.fori_loop(
                0, 30, bs, (jnp.int32(0), jnp.int32(1 << 30)))
            gt = ohe * (bits > thr).astype(jnp.float32)
            needed = C - jnp.sum(gt)
            tie = ohe * (bits == thr).astype(jnp.float32)
            tie_excl = cum_excl(tie)
            sel = gt + tie * (tie_excl < needed).astype(jnp.float32)
            slot = cum_excl(sel)
            pp = jnp.where(sel > 0, e * C + slot.astype(jnp.int32), pp)

        loss_ref[...] = jnp.reshape(E * loss / (jnp.float32(T) * T), (1, 1))
        pp_ref[...] = pp
        ei_ref[...] = idx
        g_ref[...] = gate


def _router_select(h, router_W):
    NB = T // TB
    return pl.pallas_call(
        _rs_body,
        grid=(NB + 1,),
        in_specs=[
            pl.BlockSpec((TB, D), lambda i: (jnp.minimum(i, NB - 1), 0)),
            pl.BlockSpec((D, E), lambda i: (0, 0)),
        ],
        out_specs=[
            pl.BlockSpec((TB, E), lambda i: (jnp.minimum(i, NB - 1), 0)),
            pl.BlockSpec((T // 1024, 1024), lambda i: (0, 0)),
            pl.BlockSpec((T // 1024, 1024), lambda i: (0, 0)),
            pl.BlockSpec((T // 1024, 1024), lambda i: (0, 0)),
            pl.BlockSpec((1, 1), lambda i: (0, 0)),
        ],
        out_shape=[
            jax.ShapeDtypeStruct((T, E), jnp.float32),
            jax.ShapeDtypeStruct((T // 1024, 1024), jnp.int32),
            jax.ShapeDtypeStruct((T // 1024, 1024), jnp.int32),
            jax.ShapeDtypeStruct((T // 1024, 1024), jnp.float32),
            jax.ShapeDtypeStruct((1, 1), jnp.float32),
        ],
        scratch_shapes=[pltpu.VMEM((E, T // 1024, 1024), jnp.float32)],
    )(h, router_W)


@functools.lru_cache(maxsize=None)
def _dispatch_kernel():
    return pl.kernel(
        _dispatch_body, mesh=_mesh(),
        compiler_params=pltpu.CompilerParams(needs_layout_passes=False),
        out_type=[
            jax.ShapeDtypeStruct((E * C, D), jnp.float32),
            jax.ShapeDtypeStruct((E * C, 16), jnp.float32),
            jax.ShapeDtypeStruct((E * C,), jnp.int32),
            jax.ShapeDtypeStruct((E * C,), jnp.float32),
        ],
        scratch_types=[
            pltpu.VMEM((T // 16,), jnp.int32),
            pltpu.VMEM((T // 16,), jnp.float32),
            pltpu.VMEM((SC_SLOTS,), jnp.int32),
            pltpu.VMEM((SC_SLOTS,), jnp.float32),
            pltpu.VMEM((SC_SLOTS // 128, 128), jnp.int32),
            pltpu.VMEM((SPT,), jnp.int32),
            pltpu.VMEM((SPT,), jnp.float32),
            pltpu.VMEM((SPT, 16), jnp.float32),
            pltpu.VMEM((SPT, D), jnp.float32),
            pltpu.VMEM_SHARED((SC_SLOTS,), jnp.int32),
            pltpu.VMEM_SHARED((SC_SLOTS,), jnp.float32),
            pltpu.SemaphoreType.DMA,
        ])


def _dispatch_body(pp_hbm, g_hbm, h_hbm, comp_hbm, gcomp_hbm, sid_hbm,
                   sg_hbm, pp_v, g_v, lids_v, lg_v, iota_v, seg_v, gseg_v,
                   grow_v, rows_v, ids_sh, g_sh, sem):
    c = lax.axis_index("c")
    s = lax.axis_index("s")
    wid = c * 16 + s
    tok0 = s * (T // 16)
    pltpu.sync_copy(pp_hbm.at[pl.ds(tok0, T // 16)], pp_v)
    pltpu.sync_copy(g_hbm.at[pl.ds(tok0, T // 16)], g_v)
    for g in range(SC_SLOTS // 16):
        lids_v[pl.ds(g * 16, 16)] = jnp.zeros((16,), jnp.int32)
        lg_v[pl.ds(g * 16, 16)] = jnp.zeros((16,), jnp.float32)
    for j in range(SC_SLOTS // 128):
        for g in range(8):
            iota_v[j, pl.ds(g * 16, 16)] = (
                lax.iota(jnp.int32, 16) + j * 128 + g * 16)

    @pl.when(s == 0)
    def _():
        pltpu.sync_copy(lids_v, ids_sh)
        pltpu.sync_copy(lg_v, g_sh)

    sbase = c * SC_SLOTS

    @plsc.parallel_loop(0, T // 16 // 16, unroll=8)
    def _(i):
        ppv = pp_v[pl.ds(i * 16, 16)]
        local = ppv - sbase
        match = jnp.logical_and(local >= 0, local < SC_SLOTS)
        loc = jnp.clip(local, 0, SC_SLOTS - 1)
        tok = lax.iota(jnp.int32, 16) + tok0 + i * 16
        plsc.store_scatter(lids_v, [loc], tok, mask=match)
        plsc.store_scatter(lg_v, [loc], g_v[pl.ds(i * 16, 16)], mask=match)

    plsc.subcore_barrier()
    for j in range(SC_SLOTS // 128):
        pltpu.sync_copy(lids_v.at[pl.ds(j * 128, 128)],
                        ids_sh.at[iota_v.at[j]], add=True)
        pltpu.sync_copy(lg_v.at[pl.ds(j * 128, 128)],
                        g_sh.at[iota_v.at[j]], add=True)
    plsc.subcore_barrier()
    pltpu.sync_copy(ids_sh.at[pl.ds(s * SPT, SPT)], seg_v)
    pltpu.sync_copy(g_sh.at[pl.ds(s * SPT, SPT)], gseg_v)
    pltpu.async_copy(h_hbm.at[seg_v], rows_v, sem).wait()
    pltpu.sync_copy(rows_v, comp_hbm.at[pl.ds(wid * SPT, SPT)])
    for g in range(SPT // 16):
        gv = gseg_v[pl.ds(g * 16, 16)]
        for k in range(16):
            grow_v[g * 16 + k, :] = jnp.full((16,), 1.0, jnp.float32) * gv[k]
    pltpu.sync_copy(grow_v, gcomp_hbm.at[pl.ds(wid * SPT, SPT)])
    pltpu.sync_copy(seg_v, sid_hbm.at[pl.ds(wid * SPT, SPT)])
    pltpu.sync_copy(gseg_v, sg_hbm.at[pl.ds(wid * SPT, SPT)])


def _ffn_body(x_ref, w1_ref, b1_ref, w2_ref, b2_ref, g_ref, o_ref):
    e = pl.program_id(0)
    j = pl.program_id(1)
    nj = pl.num_programs(1)

    x = x_ref[...]
    hdn = jnp.maximum(
        jnp.dot(x, w1_ref[0], preferred_element_type=jnp.float32)
        + b1_ref[0], 0.0)
    part = jnp.dot(hdn, w2_ref[0], preferred_element_type=jnp.float32)

    @pl.when(j == 0)
    def _():
        o_ref[...] = part + b2_ref[0]

    @pl.when(j > 0)
    def _():
        o_ref[...] = o_ref[...] + part

    @pl.when(j == nj - 1)
    def _():
        o_ref[...] = o_ref[...] * g_ref[:, 0:1]


def _ffn(comp, gcomp, W1, b1, W2, b2):
    HB = 768
    return pl.pallas_call(
        _ffn_body,
        grid=(E, H // HB),
        in_specs=[
            pl.BlockSpec((C, D), lambda e, j: (e, 0)),
            pl.BlockSpec((1, D, HB), lambda e, j: (e, 0, j)),
            pl.BlockSpec((1, 1, HB), lambda e, j: (e, 0, j)),
            pl.BlockSpec((1, HB, D), lambda e, j: (e, j, 0)),
            pl.BlockSpec((1, 1, D), lambda e, j: (e, 0, 0)),
            pl.BlockSpec((C, 16), lambda e, j: (e, 0)),
        ],
        out_specs=pl.BlockSpec((C, D), lambda e, j: (e, 0)),
        out_shape=jax.ShapeDtypeStruct((E * C, D), jnp.float32),
    )(comp, W1.reshape(E, D, H), b1.reshape(E, 1, H),
      W2.reshape(E, H, D), b2.reshape(E, 1, D), gcomp)


@functools.lru_cache(maxsize=None)
def _zero_out_kernel():
    return pl.kernel(
        _zero_out_body, mesh=_mesh(),
        compiler_params=pltpu.CompilerParams(needs_layout_passes=False),
        out_type=jax.ShapeDtypeStruct((T, D), jnp.float32),
        scratch_types=[pltpu.VMEM((ZR, D), jnp.float32)])


def _zero_out_body(out_hbm, z_v):
    wid = lax.axis_index("c") * 16 + lax.axis_index("s")
    base = wid * TPW

    def zrow(r, _):
        for k in range(D // 16):
            z_v[r, pl.ds(k * 16, 16)] = jnp.zeros((16,), jnp.float32)
        return 0

    lax.fori_loop(0, ZR, zrow, 0)
    for c in range(TPW // ZR):
        pltpu.sync_copy(z_v, out_hbm.at[pl.ds(base + c * ZR, ZR)])


@functools.lru_cache(maxsize=None)
def _scatter_out_kernel():
    return pl.kernel(
        _scatter_out_body, mesh=_mesh(),
        compiler_params=pltpu.CompilerParams(needs_layout_passes=False),
        out_type=[],
        scratch_types=[
            pltpu.VMEM((SPT,), jnp.int32),
            pltpu.VMEM((SPT,), jnp.float32),
            pltpu.VMEM((SPT, D), jnp.float32),
            pltpu.SemaphoreType.DMA,
        ])


def _scatter_out_body(ffn_hbm, sid_hbm, sg_hbm, out_hbm, ids_v, gs_v,
                      rows_v, sem):
    wid = lax.axis_index("c") * 16 + lax.axis_index("s")
    base = wid * SPT
    pltpu.sync_copy(sid_hbm.at[pl.ds(base, SPT)], ids_v)
    pltpu.sync_copy(sg_hbm.at[pl.ds(base, SPT)], gs_v)
    pltpu.sync_copy(ffn_hbm.at[pl.ds(base, SPT)], rows_v)
    for grp in range(SPT // 16):
        gv = gs_v[pl.ds(grp * 16, 16)]
        iv = ids_v[pl.ds(grp * 16, 16)]
        for k in range(16):
            @pl.when(gv[k] > 0)
            def _(j=grp * 16 + k, tok=iv[k]):
                pltpu.async_copy(rows_v.at[j], out_hbm.at[tok], sem)
    for grp in range(SPT // 16):
        gv = gs_v[pl.ds(grp * 16, 16)]
        iv = ids_v[pl.ds(grp * 16, 16)]
        for k in range(16):
            @pl.when(gv[k] > 0)
            def _(j=grp * 16 + k, tok=iv[k]):
                pltpu.make_async_copy(rows_v.at[j], out_hbm.at[tok], sem).wait()


def kernel(hidden_states, router_W, W1, b1, W2, b2):
    Bq, Sq, Dm = hidden_states.shape
    h = hidden_states.reshape(T, D)
    probs, pp1, ei1, g1, loss11 = _router_select(h, router_W)
    pp = pp1.reshape(T)
    gate = g1.reshape(T)
    zeros_td = _zero_out_kernel()()
    comp, gcomp, sid, sg = _dispatch_kernel()(pp, gate, h)
    ffn = _ffn(comp, gcomp, W1, b1, W2, b2)
    oref = jax.new_ref(zeros_td)
    _scatter_out_kernel()(ffn, sid, sg, oref)
    out = oref[...]
    return (out.reshape(Bq, Sq, Dm), loss11.reshape(()), probs, ei1.reshape(T))


# speedup vs baseline: 1.0180x; 1.0180x over previous
import functools

import jax
import jax.numpy as jnp
from jax import lax
from jax.experimental import pallas as pl
from jax.experimental.pallas import tpu as pltpu
from jax.experimental.pallas import tpu_sc as plsc

D = 768
E = 8
C = 128
T = 16384
H = 3072
TB = 1024
NTILES = 32
TPW = T // NTILES
CHUNK = 128
CCH = 64
SPT = (E * C) // NTILES
SC_SLOTS = (E * C) // 2
ZR = 64
DUMMY_BASE = 1032
FFN_ROWS = 1152

@functools.lru_cache(maxsize=None)
def _mesh():
    return plsc.VectorSubcoreMesh(core_axis_name="c", subcore_axis_name="s")


def _rs_body(h_ref, w_ref, probs_ref, pp_ref, ei_ref, g_ref, loss_ref, pt_ref):
    i = pl.program_id(0)
    NB = T // TB

    @pl.when(i < NB)
    def _():
        x = h_ref[...]
        w = w_ref[...]
        lt = lax.dot_general(w, x, (((0,), (1,)), ((), ())),
                             preferred_element_type=jnp.float32)
        m = jnp.max(lt, axis=0, keepdims=True)
        ex = jnp.exp(lt - m)
        pt = ex / jnp.sum(ex, axis=0, keepdims=True)
        probs_ref[...] = pt.T
        pt_ref[:, pl.ds(jnp.minimum(i, NB - 1), 1), :] = pt[:, None, :]

    @pl.when(i == NB)
    def _():
        RR, CC = 128, 128
        pe = [pt_ref[e].reshape(RR, CC) for e in range(E)]
        gate = pe[0]
        for e in range(1, E):
            gate = jnp.maximum(gate, pe[e])
        idx = jnp.full((RR, CC), E, jnp.int32)
        for e in range(E - 1, -1, -1):
            idx = jnp.where(pe[e] == gate, e, idx)

        r_iota = lax.broadcasted_iota(jnp.int32, (CC, CC), 0)
        c_iota = lax.broadcasted_iota(jnp.int32, (CC, CC), 1)
        u_inc = (r_iota <= c_iota).astype(jnp.float32)
        rs_iota = lax.broadcasted_iota(jnp.int32, (RR, RR), 0)
        cs_iota = lax.broadcasted_iota(jnp.int32, (RR, RR), 1)
        l_exc = (rs_iota > cs_iota).astype(jnp.float32)

        def cum_excl(x):
            incl = jnp.dot(x, u_inc, preferred_element_type=jnp.float32)
            rowsum = jnp.sum(x, axis=1, keepdims=True)
            off = jnp.dot(l_exc, rowsum, preferred_element_type=jnp.float32)
            return incl + off - x

        bits = lax.bitcast_convert_type(gate, jnp.int32)
        ri = lax.broadcasted_iota(jnp.int32, (RR, CC), 0)
        ci = lax.broadcasted_iota(jnp.int32, (RR, CC), 1)
        t_iota = ri * CC + ci
        pp = DUMMY_BASE + jnp.bitwise_and(t_iota, 63)
        loss = jnp.float32(0.0)
        for e in range(E):
            ohe = (idx == e).astype(jnp.float32)
            loss = loss + jnp.sum(ohe) * jnp.sum(pe[e])

            def bs(_, carry):
                lo, hi = carry
                mid = lax.div(lo + hi, jnp.int32(2))
                cnt = jnp.sum(ohe * (bits >= mid).astype(jnp.float32))
                take = cnt >= C
                return (jnp.where(take, mid, lo), jnp.where(take, hi, mid))

            thr, _ = lax.fori_loop(
                0, 30, bs, (jnp.int32(0), jnp.int32(1 << 30)))
            gt = ohe * (bits > thr).astype(jnp.float32)
            needed = C - jnp.sum(gt)
            tie = ohe * (bits == thr).astype(jnp.float32)
            tie_excl = cum_excl(tie)
            sel = gt + tie * (tie_excl < needed).astype(jnp.float32)
            slot = cum_excl(sel)
            pp = jnp.where(sel > 0, e * C + slot.astype(jnp.int32), pp)

        loss_ref[...] = jnp.reshape(E * loss / (jnp.float32(T) * T), (1, 1))
        pp_ref[...] = pp.reshape(T // 1024, 1024)
        ei_ref[...] = idx.reshape(T // 1024, 1024)
        g_ref[...] = gate.reshape(T // 1024, 1024)


def _router_select(h, router_W):
    NB = T // TB
    return pl.pallas_call(
        _rs_body,
        grid=(NB + 1,),
        in_specs=[
            pl.BlockSpec((TB, D), lambda i: (jnp.minimum(i, NB - 1), 0)),
            pl.BlockSpec((D, E), lambda i: (0, 0)),
        ],
        out_specs=[
            pl.BlockSpec((TB, E), lambda i: (jnp.minimum(i, NB - 1), 0)),
            pl.BlockSpec((T // 1024, 1024), lambda i: (0, 0)),
            pl.BlockSpec((T // 1024, 1024), lambda i: (0, 0)),
            pl.BlockSpec((T // 1024, 1024), lambda i: (0, 0)),
            pl.BlockSpec((1, 1), lambda i: (0, 0)),
        ],
        out_shape=[
            jax.ShapeDtypeStruct((T, E), jnp.float32),
            jax.ShapeDtypeStruct((T // 1024, 1024), jnp.int32),
            jax.ShapeDtypeStruct((T // 1024, 1024), jnp.int32),
            jax.ShapeDtypeStruct((T // 1024, 1024), jnp.float32),
            jax.ShapeDtypeStruct((1, 1), jnp.float32),
        ],
        scratch_shapes=[pltpu.VMEM((E, T // 1024, 1024), jnp.float32)],
    )(h, router_W)


@functools.lru_cache(maxsize=None)
def _dispatch_kernel():
    return pl.kernel(
        _dispatch_body, mesh=_mesh(),
        compiler_params=pltpu.CompilerParams(needs_layout_passes=False),
        out_type=[
            jax.ShapeDtypeStruct((E * C, D), jnp.float32),
            jax.ShapeDtypeStruct((E * C, 16), jnp.float32),
            jax.ShapeDtypeStruct((E * C,), jnp.int32),
            jax.ShapeDtypeStruct((E * C,), jnp.float32),
        ],
        scratch_types=[
            pltpu.VMEM((T // 16,), jnp.int32),
            pltpu.VMEM((T // 16,), jnp.float32),
            pltpu.VMEM((SC_SLOTS,), jnp.int32),
            pltpu.VMEM((SC_SLOTS,), jnp.float32),
            pltpu.VMEM((SC_SLOTS // 128, 128), jnp.int32),
            pltpu.VMEM((SPT,), jnp.int32),
            pltpu.VMEM((SPT,), jnp.float32),
            pltpu.VMEM((SPT, 16), jnp.float32),
            pltpu.VMEM((SPT, D), jnp.float32),
            pltpu.VMEM_SHARED((SC_SLOTS,), jnp.int32),
            pltpu.VMEM_SHARED((SC_SLOTS,), jnp.float32),
            pltpu.SemaphoreType.DMA,
        ])


def _dispatch_body(pp_hbm, g_hbm, h_hbm, comp_hbm, gcomp_hbm, sid_hbm,
                   sg_hbm, pp_v, g_v, lids_v, lg_v, iota_v, seg_v, gseg_v,
                   grow_v, rows_v, ids_sh, g_sh, sem):
    c = lax.axis_index("c")
    s = lax.axis_index("s")
    wid = c * 16 + s
    tok0 = s * (T // 16)
    pltpu.sync_copy(pp_hbm.at[pl.ds(tok0, T // 16)], pp_v)
    pltpu.sync_copy(g_hbm.at[pl.ds(tok0, T // 16)], g_v)
    for g in range(SC_SLOTS // 16):
        lids_v[pl.ds(g * 16, 16)] = jnp.zeros((16,), jnp.int32)
        lg_v[pl.ds(g * 16, 16)] = jnp.zeros((16,), jnp.float32)
    for j in range(SC_SLOTS // 128):
        for g in range(8):
            iota_v[j, pl.ds(g * 16, 16)] = (
                lax.iota(jnp.int32, 16) + j * 128 + g * 16)

    @pl.when(s == 0)
    def _():
        pltpu.sync_copy(lids_v, ids_sh)
        pltpu.sync_copy(lg_v, g_sh)

    sbase = c * SC_SLOTS

    @plsc.parallel_loop(0, T // 16 // 16, unroll=8)
    def _(i):
        ppv = pp_v[pl.ds(i * 16, 16)]
        local = ppv - sbase
        match = jnp.logical_and(local >= 0, local < SC_SLOTS)
        loc = jnp.clip(local, 0, SC_SLOTS - 1)
        tok = lax.iota(jnp.int32, 16) + tok0 + i * 16
        plsc.store_scatter(lids_v, [loc], tok, mask=match)
        plsc.store_scatter(lg_v, [loc], g_v[pl.ds(i * 16, 16)], mask=match)

    plsc.subcore_barrier()
    for j in range(SC_SLOTS // 128):
        pltpu.sync_copy(lids_v.at[pl.ds(j * 128, 128)],
                        ids_sh.at[iota_v.at[j]], add=True)
        pltpu.sync_copy(lg_v.at[pl.ds(j * 128, 128)],
                        g_sh.at[iota_v.at[j]], add=True)
    plsc.subcore_barrier()
    pltpu.sync_copy(ids_sh.at[pl.ds(s * SPT, SPT)], seg_v)
    pltpu.sync_copy(g_sh.at[pl.ds(s * SPT, SPT)], gseg_v)
    pltpu.async_copy(h_hbm.at[seg_v], rows_v, sem).wait()
    pltpu.sync_copy(rows_v, comp_hbm.at[pl.ds(wid * SPT, SPT)])
    for g in range(SPT // 16):
        gv = gseg_v[pl.ds(g * 16, 16)]
        for k in range(16):
            grow_v[g * 16 + k, :] = jnp.full((16,), 1.0, jnp.float32) * gv[k]
    pltpu.sync_copy(grow_v, gcomp_hbm.at[pl.ds(wid * SPT, SPT)])
    pltpu.sync_copy(seg_v, sid_hbm.at[pl.ds(wid * SPT, SPT)])
    pltpu.sync_copy(gseg_v, sg_hbm.at[pl.ds(wid * SPT, SPT)])


def _ffn_body(x_ref, w1_ref, b1_ref, w2_ref, b2_ref, g_ref, o_ref):
    e = pl.program_id(0)
    j = pl.program_id(1)
    nj = pl.num_programs(1)

    x = x_ref[...]
    hdn = jnp.maximum(
        jnp.dot(x, w1_ref[0], preferred_element_type=jnp.float32)
        + b1_ref[0], 0.0)
    part = jnp.dot(hdn, w2_ref[0], preferred_element_type=jnp.float32)

    @pl.when(j == 0)
    def _():
        o_ref[...] = part + b2_ref[0]

    @pl.when(j > 0)
    def _():
        o_ref[...] = o_ref[...] + part

    @pl.when(j == nj - 1)
    def _():
        o_ref[...] = o_ref[...] * g_ref[:, 0:1]


def _ffn(comp, gcomp, W1, b1, W2, b2):
    HB = 768
    return pl.pallas_call(
        _ffn_body,
        grid=(E, H // HB),
        in_specs=[
            pl.BlockSpec((C, D), lambda e, j: (e, 0)),
            pl.BlockSpec((1, D, HB), lambda e, j: (e, 0, j)),
            pl.BlockSpec((1, 1, HB), lambda e, j: (e, 0, j)),
            pl.BlockSpec((1, HB, D), lambda e, j: (e, j, 0)),
            pl.BlockSpec((1, 1, D), lambda e, j: (e, 0, 0)),
            pl.BlockSpec((C, 16), lambda e, j: (e, 0)),
        ],
        out_specs=pl.BlockSpec((C, D), lambda e, j: (e, 0)),
        out_shape=jax.ShapeDtypeStruct((E * C, D), jnp.float32),
    )(comp, W1.reshape(E, D, H), b1.reshape(E, 1, H),
      W2.reshape(E, H, D), b2.reshape(E, 1, D), gcomp)


@functools.lru_cache(maxsize=None)
def _zero_out_kernel():
    return pl.kernel(
        _zero_out_body, mesh=_mesh(),
        compiler_params=pltpu.CompilerParams(needs_layout_passes=False),
        out_type=jax.ShapeDtypeStruct((T, D), jnp.float32),
        scratch_types=[pltpu.VMEM((ZR, D), jnp.float32)])


def _zero_out_body(out_hbm, z_v):
    wid = lax.axis_index("c") * 16 + lax.axis_index("s")
    base = wid * TPW

    def zrow(r, _):
        for k in range(D // 16):
            z_v[r, pl.ds(k * 16, 16)] = jnp.zeros((16,), jnp.float32)
        return 0

    lax.fori_loop(0, ZR, zrow, 0)
    for c in range(TPW // ZR):
        pltpu.sync_copy(z_v, out_hbm.at[pl.ds(base + c * ZR, ZR)])


@functools.lru_cache(maxsize=None)
def _scatter_out_kernel():
    return pl.kernel(
        _scatter_out_body, mesh=_mesh(),
        compiler_params=pltpu.CompilerParams(needs_layout_passes=False),
        out_type=[],
        scratch_types=[
            pltpu.VMEM((SPT,), jnp.int32),
            pltpu.VMEM((SPT,), jnp.float32),
            pltpu.VMEM((SPT, D), jnp.float32),
            pltpu.SemaphoreType.DMA,
        ])


def _scatter_out_body(ffn_hbm, sid_hbm, sg_hbm, out_hbm, ids_v, gs_v,
                      rows_v, sem):
    wid = lax.axis_index("c") * 16 + lax.axis_index("s")
    base = wid * SPT
    pltpu.sync_copy(sid_hbm.at[pl.ds(base, SPT)], ids_v)
    pltpu.sync_copy(sg_hbm.at[pl.ds(base, SPT)], gs_v)
    pltpu.sync_copy(ffn_hbm.at[pl.ds(base, SPT)], rows_v)
    for grp in range(SPT // 16):
        gv = gs_v[pl.ds(grp * 16, 16)]
        iv = ids_v[pl.ds(grp * 16, 16)]
        for k in range(16):
            @pl.when(gv[k] > 0)
            def _(j=grp * 16 + k, tok=iv[k]):
                pltpu.async_copy(rows_v.at[j], out_hbm.at[tok], sem)
    for grp in range(SPT // 16):
        gv = gs_v[pl.ds(grp * 16, 16)]
        iv = ids_v[pl.ds(grp * 16, 16)]
        for k in range(16):
            @pl.when(gv[k] > 0)
            def _(j=grp * 16 + k, tok=iv[k]):
                pltpu.make_async_copy(rows_v.at[j], out_hbm.at[tok], sem).wait()


def kernel(hidden_states, router_W, W1, b1, W2, b2):
    Bq, Sq, Dm = hidden_states.shape
    h = hidden_states.reshape(T, D)
    probs, pp1, ei1, g1, loss11 = _router_select(h, router_W)
    pp = pp1.reshape(T)
    gate = g1.reshape(T)
    zeros_td = _zero_out_kernel()()
    comp, gcomp, sid, sg = _dispatch_kernel()(pp, gate, h)
    ffn = _ffn(comp, gcomp, W1, b1, W2, b2)
    oref = jax.new_ref(zeros_td)
    _scatter_out_kernel()(ffn, sid, sg, oref)
    out = oref[...]
    return (out.reshape(Bq, Sq, Dm), loss11.reshape(()), probs, ei1.reshape(T))
